# baseline (device time: 263157 ns/iter reference)
import functools

import jax
import jax.numpy as jnp
from jax import lax
from jax.experimental import pallas as pl
from jax.experimental.pallas import tpu as pltpu

N_DEV = 8
SQ = 256
SKV = 2048
HQ = 8
DH = 128
DM = 1024
SCALE = 0.08838834764831843


KT = 256
NKT = SKV // KT


def _attn_partial(c, xc, wq_ref, k_ref, v_ref, wo_ref, qbuf, sbuf, cbuf, abuf):
    qbuf[...] = lax.dot_general(
        xc, wq_ref[...], (((1,), (0,)), ((), ())),
        preferred_element_type=jnp.float32,
    )

    rows = lax.broadcasted_iota(jnp.int32, (SQ, SKV), 0) + c * SQ
    cols = lax.broadcasted_iota(jnp.int32, (SQ, SKV), 1)
    keep = (cols // 64) <= (rows // 64)

    abuf[...] = jnp.zeros((SQ, DM), jnp.float32)

    def h_body(h, carry):
        qh = qbuf[:, pl.ds(h * DH, DH)]

        def kt_body(kt, carry2):
            @pl.when(kt <= c)
            def _():
                sbuf[:, pl.ds(kt * KT, KT)] = lax.dot_general(
                    qh, k_ref[h, pl.ds(kt * KT, KT), :],
                    (((1,), (1,)), ((), ())),
                    preferred_element_type=jnp.float32,
                )
            return carry2

        lax.fori_loop(0, NKT, kt_body, 0)

        s = jnp.where(keep, sbuf[...] * SCALE, -1e9)
        m = jnp.max(s, axis=1, keepdims=True)
        w = jnp.exp(s - m)
        sbuf[...] = w / jnp.sum(w, axis=1, keepdims=True)

        cbuf[...] = jnp.zeros((SQ, DH), jnp.float32)

        def kt2_body(kt, carry2):
            @pl.when(kt <= c)
            def _():
                cbuf[...] += lax.dot_general(
                    sbuf[:, pl.ds(kt * KT, KT)],
                    v_ref[h, pl.ds(kt * KT, KT), :],
                    (((1,), (0,)), ((), ())),
                    preferred_element_type=jnp.float32,
                )
            return carry2

        lax.fori_loop(0, NKT, kt2_body, 0)

        abuf[...] += lax.dot_general(
            cbuf[...], wo_ref[pl.ds(h * DH, DH), :],
            (((1,), (0,)), ((), ())),
            preferred_element_type=jnp.float32,
        )
        return carry

    lax.fori_loop(0, HQ, h_body, 0)
    return abuf[...]


def _body(x_ref, wq_ref, k_ref, v_ref, wo_ref, out_ref,
          xbuf, rsbuf, qbuf, sbuf, cbuf, abuf,
          ag_send, ag_recv, rs_send, rs_recv):
    i = lax.axis_index("i")
    right = lax.rem(i + 1, N_DEV)

    def attn(c, xc):
        return _attn_partial(c, xc, wq_ref, k_ref, v_ref, wo_ref,
                             qbuf, sbuf, cbuf, abuf)

    def ag_rdma(t):
        return pltpu.make_async_remote_copy(
            src_ref=xbuf.at[t],
            dst_ref=xbuf.at[t + 1],
            send_sem=ag_send.at[t],
            recv_sem=ag_recv.at[t],
            device_id=(right,),
            device_id_type=pl.DeviceIdType.MESH,
        )

    def rs_rdma(t):
        slot = N_DEV - 1 if t == 0 else t - 1
        return pltpu.make_async_remote_copy(
            src_ref=rsbuf.at[slot],
            dst_ref=rsbuf.at[t],
            send_sem=rs_send.at[t],
            recv_sem=rs_recv.at[t],
            device_id=(right,),
            device_id_type=pl.DeviceIdType.MESH,
        )

    xbuf[0] = x_ref[...]
    ag_rdma(0).start()
    partial_own = attn(i, x_ref[...])

    for t in range(N_DEV - 1):
        ag_rdma(t).wait_recv()
        if t + 1 < N_DEV - 1:
            ag_rdma(t + 1).start()

        c_r = lax.rem(i - (t + 1) + N_DEV, N_DEV)
        pc = attn(c_r, xbuf[t + 1])

        if t == 0:
            rsbuf[N_DEV - 1] = pc
        else:
            rs_rdma(t - 1).wait_recv()
            rsbuf[t - 1] = rsbuf[t - 1] + pc
        rs_rdma(t).start()

    rs_rdma(N_DEV - 2).wait_recv()
    out_ref[...] = rsbuf[N_DEV - 2] + partial_own

    for t in range(N_DEV - 1):
        ag_rdma(t).wait_send()
        rs_rdma(t).wait_send()


def kernel(x, Wq, K_ext, V_ext, Wo):
    i = lax.axis_index("i")
    k_sl = lax.dynamic_slice(K_ext, (0, 0, i * HQ, 0), (1, SKV, HQ, DH))[0]
    v_sl = lax.dynamic_slice(V_ext, (0, 0, i * HQ, 0), (1, SKV, HQ, DH))[0]
    k_hm = jnp.transpose(k_sl, (1, 0, 2))
    v_hm = jnp.transpose(v_sl, (1, 0, 2))

    out = pl.pallas_call(
        _body,
        out_shape=jax.ShapeDtypeStruct((SQ, DM), jnp.float32),
        in_specs=[pl.BlockSpec(memory_space=pltpu.VMEM)] * 5,
        out_specs=pl.BlockSpec(memory_space=pltpu.VMEM),
        scratch_shapes=[
            pltpu.VMEM((N_DEV, SQ, DM), jnp.float32),
            pltpu.VMEM((N_DEV, SQ, DM), jnp.float32),
            pltpu.VMEM((SQ, HQ * DH), jnp.float32),
            pltpu.VMEM((SQ, SKV), jnp.float32),
            pltpu.VMEM((SQ, DH), jnp.float32),
            pltpu.VMEM((SQ, DM), jnp.float32),
            pltpu.SemaphoreType.DMA((N_DEV - 1,)),
            pltpu.SemaphoreType.DMA((N_DEV - 1,)),
            pltpu.SemaphoreType.DMA((N_DEV - 1,)),
            pltpu.SemaphoreType.DMA((N_DEV - 1,)),
        ],
    )(x[0], Wq, k_hm, v_hm, Wo)
    return out.reshape(1, SQ, DM)


# device time: 170424 ns/iter; 1.5441x vs baseline; 1.5441x over previous
import functools

import jax
import jax.numpy as jnp
from jax import lax
from jax.experimental import pallas as pl
from jax.experimental.pallas import tpu as pltpu

N_DEV = 8
SQ = 256
SKV = 2048
HQ = 8
DH = 128
DM = 1024
SCALE = 0.08838834764831843


def _attn_partial(c, xc, wq_ref, k_ref, v_ref, wo_ref, qbuf):
    qbuf[...] = lax.dot_general(
        xc, wq_ref[...], (((1,), (0,)), ((), ())),
        preferred_element_type=jnp.float32,
    ).astype(jnp.bfloat16)

    rows = lax.broadcasted_iota(jnp.int32, (SQ, SKV), 0) + c * SQ
    cols = lax.broadcasted_iota(jnp.int32, (SQ, SKV), 1)
    keep = (cols // 64) <= (rows // 64)

    def h_body(h, acc):
        qh = qbuf[:, pl.ds(h * DH, DH)]
        s = lax.dot_general(
            qh, k_ref[h], (((1,), (1,)), ((), ())),
            preferred_element_type=jnp.float32,
        ) * SCALE
        s = jnp.where(keep, s, -1e9)
        m = jnp.max(s, axis=1, keepdims=True)
        w = jnp.exp(s - m)
        w = (w / jnp.sum(w, axis=1, keepdims=True)).astype(jnp.bfloat16)
        ctx = lax.dot_general(
            w, v_ref[h], (((1,), (0,)), ((), ())),
            preferred_element_type=jnp.float32,
        ).astype(jnp.bfloat16)
        return acc + lax.dot_general(
            ctx, wo_ref[pl.ds(h * DH, DH), :], (((1,), (0,)), ((), ())),
            preferred_element_type=jnp.float32,
        )

    return lax.fori_loop(0, HQ, h_body, jnp.zeros((SQ, DM), jnp.float32))


def _body(x_ref, wq_ref, k_ref, v_ref, wo_ref, out_ref,
          xbuf, rsbuf, qbuf, ag_send, ag_recv, rs_send, rs_recv):
    i = lax.axis_index("i")
    right = lax.rem(i + 1, N_DEV)

    def attn(c, xc):
        return _attn_partial(c, xc, wq_ref, k_ref, v_ref, wo_ref, qbuf)

    def ag_rdma(t):
        return pltpu.make_async_remote_copy(
            src_ref=xbuf.at[t],
            dst_ref=xbuf.at[t + 1],
            send_sem=ag_send.at[t],
            recv_sem=ag_recv.at[t],
            device_id=(right,),
            device_id_type=pl.DeviceIdType.MESH,
        )

    def rs_rdma(t):
        slot = N_DEV - 1 if t == 0 else t - 1
        return pltpu.make_async_remote_copy(
            src_ref=rsbuf.at[slot],
            dst_ref=rsbuf.at[t],
            send_sem=rs_send.at[t],
            recv_sem=rs_recv.at[t],
            device_id=(right,),
            device_id_type=pl.DeviceIdType.MESH,
        )

    xbuf[0] = x_ref[...]
    ag_rdma(0).start()
    partial_own = attn(i, x_ref[...])

    for t in range(N_DEV - 1):
        ag_rdma(t).wait_recv()
        if t + 1 < N_DEV - 1:
            ag_rdma(t + 1).start()

        c_r = lax.rem(i - (t + 1) + N_DEV, N_DEV)
        pc = attn(c_r, xbuf[t + 1])

        if t == 0:
            rsbuf[N_DEV - 1] = pc
        else:
            rs_rdma(t - 1).wait_recv()
            rsbuf[t - 1] = rsbuf[t - 1] + pc
        rs_rdma(t).start()

    rs_rdma(N_DEV - 2).wait_recv()
    out_ref[...] = rsbuf[N_DEV - 2] + partial_own

    for t in range(N_DEV - 1):
        ag_rdma(t).wait_send()
        rs_rdma(t).wait_send()


def kernel(x, Wq, K_ext, V_ext, Wo):
    i = lax.axis_index("i")
    k_sl = lax.dynamic_slice(K_ext, (0, 0, i * HQ, 0), (1, SKV, HQ, DH))[0]
    v_sl = lax.dynamic_slice(V_ext, (0, 0, i * HQ, 0), (1, SKV, HQ, DH))[0]
    k_hm = jnp.transpose(k_sl, (1, 0, 2)).astype(jnp.bfloat16)
    v_hm = jnp.transpose(v_sl, (1, 0, 2)).astype(jnp.bfloat16)

    out = pl.pallas_call(
        _body,
        out_shape=jax.ShapeDtypeStruct((SQ, DM), jnp.float32),
        in_specs=[pl.BlockSpec(memory_space=pltpu.VMEM)] * 5,
        out_specs=pl.BlockSpec(memory_space=pltpu.VMEM),
        scratch_shapes=[
            pltpu.VMEM((N_DEV, SQ, DM), jnp.bfloat16),
            pltpu.VMEM((N_DEV, SQ, DM), jnp.float32),
            pltpu.VMEM((SQ, HQ * DH), jnp.bfloat16),
            pltpu.SemaphoreType.DMA((N_DEV - 1,)),
            pltpu.SemaphoreType.DMA((N_DEV - 1,)),
            pltpu.SemaphoreType.DMA((N_DEV - 1,)),
            pltpu.SemaphoreType.DMA((N_DEV - 1,)),
        ],
    )(x[0].astype(jnp.bfloat16), Wq.astype(jnp.bfloat16), k_hm, v_hm,
      Wo.astype(jnp.bfloat16))
    return out.reshape(1, SQ, DM)


# device time: 153344 ns/iter; 1.7161x vs baseline; 1.1114x over previous
import functools

import jax
import jax.numpy as jnp
from jax import lax
from jax.experimental import pallas as pl
from jax.experimental.pallas import tpu as pltpu

N_DEV = 8
SQ = 256
SKV = 2048
HQ = 8
DH = 128
DM = 1024
SCALE = 0.08838834764831843


def _attn_partial(c, xc, wq_ref, k_ref, v_ref, wo_ref, qbuf):
    qbuf[...] = lax.dot_general(
        xc, wq_ref[...], (((1,), (0,)), ((), ())),
        preferred_element_type=jnp.float32,
    ).astype(jnp.bfloat16)

    rows = lax.broadcasted_iota(jnp.int32, (SQ, SKV), 0) + c * SQ
    cols = lax.broadcasted_iota(jnp.int32, (SQ, SKV), 1)
    keep = (cols // 64) <= (rows // 64)

    def h_body(h, acc):
        qh = qbuf[:, pl.ds(h * DH, DH)]
        s = lax.dot_general(
            qh, k_ref[h], (((1,), (1,)), ((), ())),
            preferred_element_type=jnp.float32,
        ) * SCALE
        s = jnp.where(keep, s, -1e9)
        m = jnp.max(s, axis=1, keepdims=True)
        w = jnp.exp(s - m)
        w = (w / jnp.sum(w, axis=1, keepdims=True)).astype(jnp.bfloat16)
        ctx = lax.dot_general(
            w, v_ref[h], (((1,), (0,)), ((), ())),
            preferred_element_type=jnp.float32,
        ).astype(jnp.bfloat16)
        return acc + lax.dot_general(
            ctx, wo_ref[pl.ds(h * DH, DH), :], (((1,), (0,)), ((), ())),
            preferred_element_type=jnp.float32,
        )

    return lax.fori_loop(0, HQ, h_body, jnp.zeros((SQ, DM), jnp.float32))


def _body(x_ref, wq_ref, k_ref, v_ref, wo_ref, out_ref,
          xbuf, rsbuf, qbuf, ag_send, ag_recv, rs_send, rs_recv):
    i = lax.axis_index("i")
    right = lax.rem(i + 1, N_DEV)

    def attn(c, xc):
        return _attn_partial(c, xc, wq_ref, k_ref, v_ref, wo_ref, qbuf)

    def ag_rdma(t):
        return pltpu.make_async_remote_copy(
            src_ref=xbuf.at[t],
            dst_ref=xbuf.at[t + 1],
            send_sem=ag_send.at[t],
            recv_sem=ag_recv.at[t],
            device_id=(right,),
            device_id_type=pl.DeviceIdType.MESH,
        )

    def rs_rdma(t):
        slot = N_DEV - 1 if t == 0 else t - 1
        return pltpu.make_async_remote_copy(
            src_ref=rsbuf.at[slot],
            dst_ref=rsbuf.at[t],
            send_sem=rs_send.at[t],
            recv_sem=rs_recv.at[t],
            device_id=(right,),
            device_id_type=pl.DeviceIdType.MESH,
        )

    xbuf[0] = x_ref[...]
    ag_rdma(0).start()
    partial_own = attn(i, x_ref[...])

    for t in range(N_DEV - 1):
        ag_rdma(t).wait_recv()
        if t + 1 < N_DEV - 1:
            ag_rdma(t + 1).start()

        c_r = lax.rem(i - (t + 1) + N_DEV, N_DEV)
        pc = attn(c_r, xbuf[t + 1])

        if t == 0:
            rsbuf[N_DEV - 1] = pc.astype(jnp.bfloat16)
        else:
            rs_rdma(t - 1).wait_recv()
            rsbuf[t - 1] = (rsbuf[t - 1] + pc).astype(jnp.bfloat16)
        rs_rdma(t).start()

    rs_rdma(N_DEV - 2).wait_recv()
    out_ref[...] = rsbuf[N_DEV - 2] + partial_own

    for t in range(N_DEV - 1):
        ag_rdma(t).wait_send()
        rs_rdma(t).wait_send()


def kernel(x, Wq, K_ext, V_ext, Wo):
    i = lax.axis_index("i")
    k_sl = lax.dynamic_slice(K_ext, (0, 0, i * HQ, 0), (1, SKV, HQ, DH))[0]
    v_sl = lax.dynamic_slice(V_ext, (0, 0, i * HQ, 0), (1, SKV, HQ, DH))[0]
    k_hm = jnp.transpose(k_sl, (1, 0, 2)).astype(jnp.bfloat16)
    v_hm = jnp.transpose(v_sl, (1, 0, 2)).astype(jnp.bfloat16)

    out = pl.pallas_call(
        _body,
        out_shape=jax.ShapeDtypeStruct((SQ, DM), jnp.float32),
        in_specs=[pl.BlockSpec(memory_space=pltpu.VMEM)] * 5,
        out_specs=pl.BlockSpec(memory_space=pltpu.VMEM),
        scratch_shapes=[
            pltpu.VMEM((N_DEV, SQ, DM), jnp.bfloat16),
            pltpu.VMEM((N_DEV, SQ, DM), jnp.bfloat16),
            pltpu.VMEM((SQ, HQ * DH), jnp.bfloat16),
            pltpu.SemaphoreType.DMA((N_DEV - 1,)),
            pltpu.SemaphoreType.DMA((N_DEV - 1,)),
            pltpu.SemaphoreType.DMA((N_DEV - 1,)),
            pltpu.SemaphoreType.DMA((N_DEV - 1,)),
        ],
    )(x[0].astype(jnp.bfloat16), Wq.astype(jnp.bfloat16), k_hm, v_hm,
      Wo.astype(jnp.bfloat16))
    return out.reshape(1, SQ, DM)
